# baseline (speedup 1.0000x reference)
import functools

import jax
import jax.numpy as jnp
from jax import lax
from jax.experimental import pallas as pl
from jax.experimental.pallas import tpu as pltpu
from jax.experimental.pallas import tpu_sc as plsc

N = 10000
D = 128
E = 320000

NC = 2
NS = 16
NW = NC * NS

CHUNK = 128
CH = 80
EPT = CH * CHUNK
EPAD = NW * EPT
ACC_ROWS = 10112
DUMMY = N
RPT = ACC_ROWS // NS

_MESH = plsc.VectorSubcoreMesh(
    core_axis_name="c", subcore_axis_name="s", num_cores=NC, num_subcores=NS
)


CHT = NW * CH // NS


_TILE_SLICES = [(0, 128), (128, 128), (256, 128), (384, 128), (512, 120)]


def _zero_tile_rows(z128, ref, base):
    for off, ln in _TILE_SLICES:
        pltpu.sync_copy(z128.at[pl.ds(0, ln)], ref.at[pl.ds(base + off, ln)])


def _copyout_tile_rows(ref, out, cid, base):
    for off, ln in _TILE_SLICES:
        pltpu.sync_copy(ref.at[pl.ds(base + off, ln)],
                        out.at[cid, pl.ds(base + off, ln)])


def _deg_body(idxall, e1_hbm, z128, out, cnt, idx_v, e1_v, wsem):
    cid = lax.axis_index("c")
    sid = lax.axis_index("s")
    _zero_tile_rows(z128, cnt, sid * RPT)
    pltpu.sync_copy(e1_hbm, e1_v)
    pltpu.sync_copy(idxall.at[pl.ds(cid * (NW * CH) + sid * CHT, CHT)], idx_v)
    plsc.subcore_barrier()

    def body(g, carry):
        for q in range(32):
            pltpu.async_copy(e1_v, cnt.at[idx_v.at[g * 32 + q]], wsem,
                             add=True)
        for q in range(32):
            pltpu.make_async_copy(e1_v, cnt.at[idx_v.at[0]], wsem).wait()
        return carry

    lax.fori_loop(0, CHT // 32, body, 0)
    plsc.subcore_barrier()
    _copyout_tile_rows(cnt, out, cid, sid * RPT)


def _make_deg_kernel(interpret=False):
    return pl.kernel(
        _deg_body,
        out_type=jax.ShapeDtypeStruct((NC, ACC_ROWS, D), jnp.float32),
        mesh=_MESH,
        scratch_types=[
            pltpu.VMEM_SHARED((ACC_ROWS, D), jnp.float32),
            pltpu.VMEM((CHT, CHUNK), jnp.int32),
            pltpu.VMEM((CHUNK, D), jnp.float32),
            pltpu.SemaphoreType.DMA,
        ],
        interpret=interpret,
    )


_deg_kernel = _make_deg_kernel()


NBUF = 2
PH = 40
SLOW_CID = 0
FAST_PH = 4
SLOW_PH = 2 * CH // PH - FAST_PH


def _edge_body(table, srcp, dstp, z128, out, acc, src_v, dst_v, rows_v,
               gsem, ssem):
    cid = lax.axis_index("c")
    sid = lax.axis_index("s")
    _zero_tile_rows(z128, acc, sid * RPT)

    def gather(j, b):
        pltpu.async_copy(table.at[src_v.at[j]], rows_v.at[b], gsem.at[b])

    def wait_gather(b):
        pltpu.make_async_copy(table.at[src_v.at[0]], rows_v.at[b],
                              gsem.at[b]).wait()

    def scatter(j, b):
        pltpu.async_copy(rows_v.at[b], acc.at[dst_v.at[j]], ssem.at[b],
                         add=True)

    def wait_scatter(b):
        pltpu.make_async_copy(rows_v.at[b], acc.at[dst_v.at[0]],
                              ssem.at[b]).wait()

    nph = jnp.where(cid == SLOW_CID, SLOW_PH, FAST_PH)
    cb = jnp.where(cid == SLOW_CID,
                   NS * FAST_PH * PH + sid * (SLOW_PH * PH),
                   sid * (FAST_PH * PH))

    plsc.subcore_barrier()

    def phase(p, carry):
        base = cb + p * PH
        pltpu.sync_copy(srcp.at[pl.ds(base, PH)], src_v)
        pltpu.sync_copy(dstp.at[pl.ds(base, PH)], dst_v)
        for b in range(NBUF):
            gather(b, b)

        def body(g, c2):
            for b in range(NBUF):
                wait_gather(b)
                scatter(g * NBUF + b, b)
            for b in range(NBUF):
                wait_scatter(b)
                gather((g + 1) * NBUF + b, b)
            return c2

        lax.fori_loop(0, PH // NBUF - 1, body, 0)
        for b in range(NBUF):
            wait_gather(b)
            scatter(PH - NBUF + b, b)
        for b in range(NBUF):
            wait_scatter(b)
        return carry

    lax.fori_loop(0, nph, phase, 0)
    plsc.subcore_barrier()
    _copyout_tile_rows(acc, out, cid, sid * RPT)


def _make_edge_kernel(interpret=False):
    return pl.kernel(
        _edge_body,
        out_type=jax.ShapeDtypeStruct((NC, ACC_ROWS, D), jnp.float32),
        mesh=_MESH,
        scratch_types=[
            pltpu.VMEM_SHARED((ACC_ROWS, D), jnp.float32),
            pltpu.VMEM((PH, CHUNK), jnp.int32),
            pltpu.VMEM((PH, CHUNK), jnp.int32),
            pltpu.VMEM((NBUF, CHUNK, D), jnp.float32),
            pltpu.SemaphoreType.DMA((NBUF,)),
            pltpu.SemaphoreType.DMA((NBUF,)),
        ],
        interpret=interpret,
    )


_edge_kernel = _make_edge_kernel()


_BLK = 2000
_GRID = N // _BLK


def _rsqrt_col0(cnt):
    return lax.rsqrt(jnp.maximum(cnt[:, 0:1], 1.0))


def _tc0_body(x_ref, w_ref, oc_ref, o_ref):
    t = jnp.dot(x_ref[...], w_ref[...], preferred_element_type=jnp.float32,
                precision=lax.Precision.HIGHEST)
    o_ref[...] = t * _rsqrt_col0(oc_ref[...])


_tc0 = pl.pallas_call(
    _tc0_body,
    grid=(_GRID,),
    in_specs=[
        pl.BlockSpec((_BLK, D), lambda i: (i, 0)),
        pl.BlockSpec((D, D), lambda i: (0, 0)),
        pl.BlockSpec((_BLK, D), lambda i: (i, 0)),
    ],
    out_specs=pl.BlockSpec((_BLK, D), lambda i: (i, 0)),
    out_shape=jax.ShapeDtypeStruct((N, D), jnp.float32),
)


def _tcmid_body(p0_ref, p1_ref, ic_ref, b_ref, oc_ref, w_ref, o_ref):
    x = (p0_ref[...] + p1_ref[...]) * _rsqrt_col0(ic_ref[...]) + b_ref[...]
    x = jnp.maximum(x, 0.0)
    t = jnp.dot(x, w_ref[...], preferred_element_type=jnp.float32,
                precision=lax.Precision.HIGHEST)
    o_ref[...] = t * _rsqrt_col0(oc_ref[...])


_tcmid = pl.pallas_call(
    _tcmid_body,
    grid=(_GRID,),
    in_specs=[
        pl.BlockSpec((_BLK, D), lambda i: (i, 0)),
        pl.BlockSpec((_BLK, D), lambda i: (i, 0)),
        pl.BlockSpec((_BLK, D), lambda i: (i, 0)),
        pl.BlockSpec((1, D), lambda i: (0, 0)),
        pl.BlockSpec((_BLK, D), lambda i: (i, 0)),
        pl.BlockSpec((D, D), lambda i: (0, 0)),
    ],
    out_specs=pl.BlockSpec((_BLK, D), lambda i: (i, 0)),
    out_shape=jax.ShapeDtypeStruct((N, D), jnp.float32),
)


def _tcfin_body(p0_ref, p1_ref, ic_ref, b_ref, o_ref):
    o_ref[...] = ((p0_ref[...] + p1_ref[...]) * _rsqrt_col0(ic_ref[...])
                  + b_ref[...])


_tcfin = pl.pallas_call(
    _tcfin_body,
    grid=(_GRID,),
    in_specs=[
        pl.BlockSpec((_BLK, D), lambda i: (i, 0)),
        pl.BlockSpec((_BLK, D), lambda i: (i, 0)),
        pl.BlockSpec((_BLK, D), lambda i: (i, 0)),
        pl.BlockSpec((1, D), lambda i: (0, 0)),
    ],
    out_specs=pl.BlockSpec((_BLK, D), lambda i: (i, 0)),
    out_shape=jax.ShapeDtypeStruct((N, D), jnp.float32),
)


def kernel(features, edge_index, W0, b0, W1, b1, W2, b2):
    src = edge_index[0]
    dst = edge_index[1]
    pad = EPAD - E
    srcg = jnp.concatenate([src, jnp.zeros((pad,), jnp.int32)]).reshape(NW * CH, CHUNK)
    srcc = jnp.concatenate([src, jnp.full((pad,), DUMMY, jnp.int32)]).reshape(NW * CH, CHUNK)
    dstp = jnp.concatenate([dst, jnp.full((pad,), DUMMY, jnp.int32)]).reshape(NW * CH, CHUNK)
    e1 = jnp.zeros((CHUNK, D), jnp.float32).at[:, 0].set(1.0)
    z128 = jnp.zeros((CHUNK, D), jnp.float32)

    cnts = _deg_kernel(jnp.concatenate([srcc, dstp]), e1, z128)
    oc = cnts[0]
    ic = cnts[1]

    t = _tc0(features, W0, oc)
    p = _edge_kernel(t, srcg, dstp, z128)
    t = _tcmid(p[0], p[1], ic, b0.reshape(1, D), oc, W1)
    p = _edge_kernel(t, srcg, dstp, z128)
    t = _tcmid(p[0], p[1], ic, b1.reshape(1, D), oc, W2)
    p = _edge_kernel(t, srcg, dstp, z128)
    return _tcfin(p[0], p[1], ic, b2.reshape(1, D))


# speedup vs baseline: 1.2434x; 1.2434x over previous
import functools

import jax
import jax.numpy as jnp
from jax import lax
from jax.experimental import pallas as pl
from jax.experimental.pallas import tpu as pltpu
from jax.experimental.pallas import tpu_sc as plsc

N = 10000
D = 128
E = 320000

NC = 2
NS = 16
NW = NC * NS

CHUNK = 128
CH = 80
EPT = CH * CHUNK
EPAD = NW * EPT
ACC_ROWS = 10112
DUMMY = N
RPT = ACC_ROWS // NS

_MESH = plsc.VectorSubcoreMesh(
    core_axis_name="c", subcore_axis_name="s", num_cores=NC, num_subcores=NS
)


CHT = NW * CH // NS


_TILE_SLICES = [(0, 128), (128, 128), (256, 128), (384, 128), (512, 120)]


def _zero_tile_rows(z128, ref, base):
    for off, ln in _TILE_SLICES:
        pltpu.sync_copy(z128.at[pl.ds(0, ln)], ref.at[pl.ds(base + off, ln)])


def _copyout_tile_rows(ref, out, cid, base):
    for off, ln in _TILE_SLICES:
        pltpu.sync_copy(ref.at[pl.ds(base + off, ln)],
                        out.at[cid, pl.ds(base + off, ln)])


def _deg_body(idxall, e1_hbm, z128, out, cnt, idx_v, e1_v, wsem):
    cid = lax.axis_index("c")
    sid = lax.axis_index("s")
    _zero_tile_rows(z128, cnt, sid * RPT)
    pltpu.sync_copy(e1_hbm, e1_v)
    pltpu.sync_copy(idxall.at[pl.ds(cid * (NW * CH) + sid * CHT, CHT)], idx_v)
    plsc.subcore_barrier()

    def body(g, carry):
        for q in range(32):
            pltpu.async_copy(e1_v, cnt.at[idx_v.at[g * 32 + q]], wsem,
                             add=True)
        for q in range(32):
            pltpu.make_async_copy(e1_v, cnt.at[idx_v.at[0]], wsem).wait()
        return carry

    lax.fori_loop(0, CHT // 32, body, 0)
    plsc.subcore_barrier()
    _copyout_tile_rows(cnt, out, cid, sid * RPT)


def _make_deg_kernel(interpret=False):
    return pl.kernel(
        _deg_body,
        out_type=jax.ShapeDtypeStruct((NC, ACC_ROWS, D), jnp.float32),
        mesh=_MESH,
        scratch_types=[
            pltpu.VMEM_SHARED((ACC_ROWS, D), jnp.float32),
            pltpu.VMEM((CHT, CHUNK), jnp.int32),
            pltpu.VMEM((CHUNK, D), jnp.float32),
            pltpu.SemaphoreType.DMA,
        ],
        interpret=interpret,
    )


_deg_kernel = _make_deg_kernel()


NBUF = 2
PH = 40
SLOW_CID = 0
FAST_PH = 3
SLOW_PH = 2 * CH // PH - FAST_PH


def _edge_body(table, srcp, dstp, z128, out, acc, src_v, dst_v, rows_v,
               gsem, ssem):
    cid = lax.axis_index("c")
    sid = lax.axis_index("s")
    _zero_tile_rows(z128, acc, sid * RPT)

    def gather(j, b):
        pltpu.async_copy(table.at[src_v.at[j]], rows_v.at[b], gsem.at[b])

    def wait_gather(b):
        pltpu.make_async_copy(table.at[src_v.at[0]], rows_v.at[b],
                              gsem.at[b]).wait()

    def scatter(j, b):
        pltpu.async_copy(rows_v.at[b], acc.at[dst_v.at[j]], ssem.at[b],
                         add=True)

    def wait_scatter(b):
        pltpu.make_async_copy(rows_v.at[b], acc.at[dst_v.at[0]],
                              ssem.at[b]).wait()

    nph = jnp.where(cid == SLOW_CID, SLOW_PH, FAST_PH)
    cb = jnp.where(cid == SLOW_CID,
                   NS * FAST_PH * PH + sid * (SLOW_PH * PH),
                   sid * (FAST_PH * PH))

    plsc.subcore_barrier()

    def phase(p, carry):
        base = cb + p * PH
        pltpu.sync_copy(srcp.at[pl.ds(base, PH)], src_v)
        pltpu.sync_copy(dstp.at[pl.ds(base, PH)], dst_v)
        for b in range(NBUF):
            gather(b, b)

        def body(g, c2):
            for b in range(NBUF):
                wait_gather(b)
                scatter(g * NBUF + b, b)
            for b in range(NBUF):
                wait_scatter(b)
                gather((g + 1) * NBUF + b, b)
            return c2

        lax.fori_loop(0, PH // NBUF - 1, body, 0)
        for b in range(NBUF):
            wait_gather(b)
            scatter(PH - NBUF + b, b)
        for b in range(NBUF):
            wait_scatter(b)
        return carry

    lax.fori_loop(0, nph, phase, 0)
    plsc.subcore_barrier()
    _copyout_tile_rows(acc, out, cid, sid * RPT)


def _make_edge_kernel(interpret=False):
    return pl.kernel(
        _edge_body,
        out_type=jax.ShapeDtypeStruct((NC, ACC_ROWS, D), jnp.float32),
        mesh=_MESH,
        scratch_types=[
            pltpu.VMEM_SHARED((ACC_ROWS, D), jnp.float32),
            pltpu.VMEM((PH, CHUNK), jnp.int32),
            pltpu.VMEM((PH, CHUNK), jnp.int32),
            pltpu.VMEM((NBUF, CHUNK, D), jnp.float32),
            pltpu.SemaphoreType.DMA((NBUF,)),
            pltpu.SemaphoreType.DMA((NBUF,)),
        ],
        interpret=interpret,
    )


_edge_kernel = _make_edge_kernel()


_BLK = 2000
_GRID = N // _BLK


def _rsqrt_col0(cnt):
    return lax.rsqrt(jnp.maximum(cnt[:, 0:1], 1.0))


def _tc0_body(x_ref, w_ref, oc_ref, o_ref):
    t = jnp.dot(x_ref[...], w_ref[...], preferred_element_type=jnp.float32,
                precision=lax.Precision.HIGHEST)
    o_ref[...] = t * _rsqrt_col0(oc_ref[...])


_tc0 = pl.pallas_call(
    _tc0_body,
    grid=(_GRID,),
    in_specs=[
        pl.BlockSpec((_BLK, D), lambda i: (i, 0)),
        pl.BlockSpec((D, D), lambda i: (0, 0)),
        pl.BlockSpec((_BLK, D), lambda i: (i, 0)),
    ],
    out_specs=pl.BlockSpec((_BLK, D), lambda i: (i, 0)),
    out_shape=jax.ShapeDtypeStruct((N, D), jnp.float32),
)


def _tcmid_body(p0_ref, p1_ref, ic_ref, b_ref, oc_ref, w_ref, o_ref):
    x = (p0_ref[...] + p1_ref[...]) * _rsqrt_col0(ic_ref[...]) + b_ref[...]
    x = jnp.maximum(x, 0.0)
    t = jnp.dot(x, w_ref[...], preferred_element_type=jnp.float32,
                precision=lax.Precision.HIGHEST)
    o_ref[...] = t * _rsqrt_col0(oc_ref[...])


_tcmid = pl.pallas_call(
    _tcmid_body,
    grid=(_GRID,),
    in_specs=[
        pl.BlockSpec((_BLK, D), lambda i: (i, 0)),
        pl.BlockSpec((_BLK, D), lambda i: (i, 0)),
        pl.BlockSpec((_BLK, D), lambda i: (i, 0)),
        pl.BlockSpec((1, D), lambda i: (0, 0)),
        pl.BlockSpec((_BLK, D), lambda i: (i, 0)),
        pl.BlockSpec((D, D), lambda i: (0, 0)),
    ],
    out_specs=pl.BlockSpec((_BLK, D), lambda i: (i, 0)),
    out_shape=jax.ShapeDtypeStruct((N, D), jnp.float32),
)


def _tcfin_body(p0_ref, p1_ref, ic_ref, b_ref, o_ref):
    o_ref[...] = ((p0_ref[...] + p1_ref[...]) * _rsqrt_col0(ic_ref[...])
                  + b_ref[...])


_tcfin = pl.pallas_call(
    _tcfin_body,
    grid=(_GRID,),
    in_specs=[
        pl.BlockSpec((_BLK, D), lambda i: (i, 0)),
        pl.BlockSpec((_BLK, D), lambda i: (i, 0)),
        pl.BlockSpec((_BLK, D), lambda i: (i, 0)),
        pl.BlockSpec((1, D), lambda i: (0, 0)),
    ],
    out_specs=pl.BlockSpec((_BLK, D), lambda i: (i, 0)),
    out_shape=jax.ShapeDtypeStruct((N, D), jnp.float32),
)


def kernel(features, edge_index, W0, b0, W1, b1, W2, b2):
    src = edge_index[0]
    dst = edge_index[1]
    pad = EPAD - E
    srcg = jnp.concatenate([src, jnp.zeros((pad,), jnp.int32)]).reshape(NW * CH, CHUNK)
    srcc = jnp.concatenate([src, jnp.full((pad,), DUMMY, jnp.int32)]).reshape(NW * CH, CHUNK)
    dstp = jnp.concatenate([dst, jnp.full((pad,), DUMMY, jnp.int32)]).reshape(NW * CH, CHUNK)
    e1 = jnp.zeros((CHUNK, D), jnp.float32).at[:, 0].set(1.0)
    z128 = jnp.zeros((CHUNK, D), jnp.float32)

    cnts = _deg_kernel(jnp.concatenate([srcc, dstp]), e1, z128)
    oc = cnts[0]
    ic = cnts[1]

    t = _tc0(features, W0, oc)
    p = _edge_kernel(t, srcg, dstp, z128)
    t = _tcmid(p[0], p[1], ic, b0.reshape(1, D), oc, W1)
    p = _edge_kernel(t, srcg, dstp, z128)
    t = _tcmid(p[0], p[1], ic, b1.reshape(1, D), oc, W2)
    p = _edge_kernel(t, srcg, dstp, z128)
    return _tcfin(p[0], p[1], ic, b2.reshape(1, D))
